# baseline (speedup 1.0000x reference)
import jax
import jax.numpy as jnp
from jax.experimental import pallas as pl
from jax.experimental.pallas import tpu as pltpu

NEG = -1e9
H = 128
G = 2


def _norm_table_kernel(emb_ref, out_ref):
    x = emb_ref[...]
    ssq = jnp.sum(x * x, axis=1, keepdims=True)
    out_ref[...] = (x * jax.lax.rsqrt(ssq)).reshape(x.shape[0], 1, H)


def _gather_rows(emb_t, ext_t, idx_ref, emb_slot, ext_slot, g, base, n):
    for mi in range(n):
        idx = idx_ref[g, 0, mi]
        emb_slot[pl.ds(base + mi, 1)] = emb_t[pl.ds(idx, 1)]
        ext_slot[pl.ds(base + mi, 1)] = ext_t[pl.ds(idx, 1)]


def _attention(td_ref, items_row, corr_col, gh, gn, hx, nx, out_ref):
    S = gh.shape[0]
    scores = jax.lax.dot_general(gn, gh, (((1,), (1,)), ((), ())),
                                 preferred_element_type=jnp.float32)
    q_iota = jax.lax.broadcasted_iota(jnp.int32, (S, S), 0)
    s_iota = jax.lax.broadcasted_iota(jnp.int32, (S, S), 1)
    causal = jnp.where(s_iota > q_iota, NEG, 0.0)
    pad_row = jnp.where(items_row == 0, NEG, 0.0)
    bias = jnp.minimum(pad_row, causal)
    dist = (q_iota + 1 - s_iota).astype(jnp.float32)
    logits = scores + bias + td_ref[0] * dist + td_ref[1]
    m = jnp.max(logits, axis=1, keepdims=True)
    e = jnp.exp(logits - m)

    rv0 = hx[:, 1:2]
    rv1 = hx[:, 2:3]
    mast = jnp.where(corr_col == 2, rv1, rv0)
    w2 = jnp.concatenate([mast, jnp.ones_like(mast)], axis=1)
    nd = jnp.dot(e, w2, preferred_element_type=jnp.float32)
    ability = nd[:, 0:1] / nd[:, 1:2]
    beta = nx[:, 0:1]
    out_ref[...] = jax.nn.sigmoid(ability - beta)


def _dpfa_kernel(td_ref, emb_t, ext_t, hist_sref, next_sref,
                 items_ref, corr_ref, out_ref, *scratches):
    S = items_ref.shape[2]
    for g in range(G):
        he_ref, hx_ref, ne_ref, nx_ref = scratches[4 * g:4 * g + 4]
        _gather_rows(emb_t, ext_t, hist_sref, he_ref, hx_ref, g, 0, S)
        _gather_rows(emb_t, ext_t, next_sref, ne_ref, nx_ref, g, 0, S)
        gh = he_ref[...].reshape(S, H)
        gn = ne_ref[...].reshape(S, H)
        hx = hx_ref[...].reshape(S, 128)
        nx = nx_ref[...].reshape(S, 128)
        _attention(td_ref, items_ref[g], corr_ref[g], gh, gn, hx, nx,
                   out_ref.at[g])


def kernel(history_items, next_items, history_corrects, item_embedding,
           item_beta_weights, item_response_vals, td_kernel, td_bias):
    B, S = history_items.shape
    V = item_embedding.shape[0]

    extras = jnp.concatenate(
        [item_beta_weights[:, None], item_response_vals,
         jnp.zeros((V, 125), dtype=jnp.float32)], axis=1).reshape(V, 1, 128)

    emb_t = pl.pallas_call(
        _norm_table_kernel,
        grid=(10,),
        in_specs=[pl.BlockSpec((V // 10, H), lambda i: (i, 0))],
        out_specs=pl.BlockSpec((V // 10, 1, H), lambda i: (i, 0, 0)),
        out_shape=jax.ShapeDtypeStruct((V, 1, H), jnp.float32),
        compiler_params=pltpu.CompilerParams(
            dimension_semantics=("parallel",)),
    )(item_embedding)

    td = jnp.concatenate([td_kernel, td_bias]).astype(jnp.float32)
    hist_i = history_items.astype(jnp.int32).reshape(B, 1, S)
    next_i = next_items.astype(jnp.int32).reshape(B, 1, S)
    corr_c = history_corrects.astype(jnp.int32).reshape(B, S, 1)

    out = pl.pallas_call(
        _dpfa_kernel,
        grid=(B // G,),
        in_specs=[
            pl.BlockSpec(memory_space=pltpu.SMEM),
            pl.BlockSpec((V, 1, H), lambda b: (0, 0, 0)),
            pl.BlockSpec((V, 1, 128), lambda b: (0, 0, 0)),
            pl.BlockSpec((G, 1, S), lambda b: (b, 0, 0),
                         memory_space=pltpu.SMEM),
            pl.BlockSpec((G, 1, S), lambda b: (b, 0, 0),
                         memory_space=pltpu.SMEM),
            pl.BlockSpec((G, 1, S), lambda b: (b, 0, 0)),
            pl.BlockSpec((G, S, 1), lambda b: (b, 0, 0)),
        ],
        out_specs=pl.BlockSpec((G, S, 1), lambda b: (b, 0, 0)),
        out_shape=jax.ShapeDtypeStruct((B, S, 1), jnp.float32),
        scratch_shapes=[pltpu.VMEM((S, 1, 128), jnp.float32)
                        for _ in range(4 * G)],
        compiler_params=pltpu.CompilerParams(
            dimension_semantics=("parallel",)),
    )(td, emb_t, extras, hist_i, next_i, hist_i, corr_c)
    return out.reshape(B, S)


# speedup vs baseline: 1.0235x; 1.0235x over previous
import jax
import jax.numpy as jnp
from jax.experimental import pallas as pl
from jax.experimental.pallas import tpu as pltpu

NEG = -1e9
H = 128
G = 4


def _norm_table_kernel(emb_ref, out_ref):
    x = emb_ref[...]
    ssq = jnp.sum(x * x, axis=1, keepdims=True)
    out_ref[...] = (x * jax.lax.rsqrt(ssq)).reshape(x.shape[0], 1, H)


def _gather_rows(emb_t, ext_t, idx_ref, emb_slot, ext_slot, g, base, n):
    for mi in range(n):
        idx = idx_ref[g, 0, mi]
        emb_slot[pl.ds(base + mi, 1)] = emb_t[pl.ds(idx, 1)]
        ext_slot[pl.ds(base + mi, 1)] = ext_t[pl.ds(idx, 1)]


def _attention(td_ref, items_row, corr_col, gh, gn, hx, nx, out_ref):
    S = gh.shape[0]
    scores = jax.lax.dot_general(gn, gh, (((1,), (1,)), ((), ())),
                                 preferred_element_type=jnp.float32)
    q_iota = jax.lax.broadcasted_iota(jnp.int32, (S, S), 0)
    s_iota = jax.lax.broadcasted_iota(jnp.int32, (S, S), 1)
    causal = jnp.where(s_iota > q_iota, NEG, 0.0)
    pad_row = jnp.where(items_row == 0, NEG, 0.0)
    bias = jnp.minimum(pad_row, causal)
    dist = (q_iota + 1 - s_iota).astype(jnp.float32)
    logits = scores + bias + td_ref[0] * dist + td_ref[1]
    m = jnp.max(logits, axis=1, keepdims=True)
    e = jnp.exp(logits - m)

    rv0 = hx[:, 1:2]
    rv1 = hx[:, 2:3]
    mast = jnp.where(corr_col == 2, rv1, rv0)
    w2 = jnp.concatenate([mast, jnp.ones_like(mast)], axis=1)
    nd = jnp.dot(e, w2, preferred_element_type=jnp.float32)
    ability = nd[:, 0:1] / nd[:, 1:2]
    beta = nx[:, 0:1]
    out_ref[...] = jax.nn.sigmoid(ability - beta)


def _dpfa_kernel(td_ref, emb_t, ext_t, hist_sref, next_sref,
                 items_ref, corr_ref, out_ref, *scratches):
    S = items_ref.shape[2]
    for g in range(G):
        he_ref, hx_ref, ne_ref, nx_ref = scratches[4 * g:4 * g + 4]
        _gather_rows(emb_t, ext_t, hist_sref, he_ref, hx_ref, g, 0, S)
        _gather_rows(emb_t, ext_t, next_sref, ne_ref, nx_ref, g, 0, S)
        gh = he_ref[...].reshape(S, H)
        gn = ne_ref[...].reshape(S, H)
        hx = hx_ref[...].reshape(S, 128)
        nx = nx_ref[...].reshape(S, 128)
        _attention(td_ref, items_ref[g], corr_ref[g], gh, gn, hx, nx,
                   out_ref.at[g])


def kernel(history_items, next_items, history_corrects, item_embedding,
           item_beta_weights, item_response_vals, td_kernel, td_bias):
    B, S = history_items.shape
    V = item_embedding.shape[0]

    extras = jnp.concatenate(
        [item_beta_weights[:, None], item_response_vals,
         jnp.zeros((V, 125), dtype=jnp.float32)], axis=1).reshape(V, 1, 128)

    emb_t = pl.pallas_call(
        _norm_table_kernel,
        grid=(10,),
        in_specs=[pl.BlockSpec((V // 10, H), lambda i: (i, 0))],
        out_specs=pl.BlockSpec((V // 10, 1, H), lambda i: (i, 0, 0)),
        out_shape=jax.ShapeDtypeStruct((V, 1, H), jnp.float32),
        compiler_params=pltpu.CompilerParams(
            dimension_semantics=("parallel",)),
    )(item_embedding)

    td = jnp.concatenate([td_kernel, td_bias]).astype(jnp.float32)
    hist_i = history_items.astype(jnp.int32).reshape(B, 1, S)
    next_i = next_items.astype(jnp.int32).reshape(B, 1, S)
    corr_c = history_corrects.astype(jnp.int32).reshape(B, S, 1)

    out = pl.pallas_call(
        _dpfa_kernel,
        grid=(B // G,),
        in_specs=[
            pl.BlockSpec(memory_space=pltpu.SMEM),
            pl.BlockSpec((V, 1, H), lambda b: (0, 0, 0)),
            pl.BlockSpec((V, 1, 128), lambda b: (0, 0, 0)),
            pl.BlockSpec((G, 1, S), lambda b: (b, 0, 0),
                         memory_space=pltpu.SMEM),
            pl.BlockSpec((G, 1, S), lambda b: (b, 0, 0),
                         memory_space=pltpu.SMEM),
            pl.BlockSpec((G, 1, S), lambda b: (b, 0, 0)),
            pl.BlockSpec((G, S, 1), lambda b: (b, 0, 0)),
        ],
        out_specs=pl.BlockSpec((G, S, 1), lambda b: (b, 0, 0)),
        out_shape=jax.ShapeDtypeStruct((B, S, 1), jnp.float32),
        scratch_shapes=[pltpu.VMEM((S, 1, 128), jnp.float32)
                        for _ in range(4 * G)],
        compiler_params=pltpu.CompilerParams(
            dimension_semantics=("parallel",)),
    )(td, emb_t, extras, hist_i, next_i, hist_i, corr_c)
    return out.reshape(B, S)
